# baseline (speedup 1.0000x reference)
import functools
import jax
import jax.numpy as jnp
from jax import lax
from jax.experimental import pallas as pl
from jax.experimental.pallas import tpu as pltpu
from jax.experimental.pallas import tpu_sc as plsc

_N = 10000
_E = 320000
_D = 128
_NC = 2
_NS = 16
_NW = _NC * _NS
_EPT = _E // _NW
_CH = 128
_NCHUNK = 80
_EPAD = _NCHUNK * _CH
_ACC_ROWS = 10240
_ZPT = _ACC_ROWS // _NS
_RPT = _ZPT

_mesh = plsc.VectorSubcoreMesh(core_axis_name="c", subcore_axis_name="s")


def _zero_rows(rows):
    @pl.loop(0, _CH)
    def _(i):
        for j in range(_D // 16):
            rows[i, pl.ds(j * 16, 16)] = jnp.zeros((16,), jnp.float32)


def _zero_acc_slice(rows, acc, s):
    base = s * _ZPT
    nfull = _ZPT // _CH
    for k in range(nfull):
        pltpu.sync_copy(rows, acc.at[pl.ds(base + k * _CH, _CH)])
    rem = _ZPT - nfull * _CH
    if rem:
        pltpu.sync_copy(rows.at[pl.ds(0, rem)], acc.at[pl.ds(base + nfull * _CH, rem)])


@functools.partial(
    pl.kernel,
    mesh=_mesh,
    out_type=jax.ShapeDtypeStruct((_NC, _ACC_ROWS, _D), jnp.float32),
    scratch_types=[
        pltpu.VMEM((_NCHUNK, _CH), jnp.int32),
        pltpu.VMEM((_CH, _D), jnp.float32),
        pltpu.VMEM_SHARED((_ACC_ROWS, _D), jnp.float32),
    ],
)
def _deg_kernel(dst_hbm, out_hbm, dst_v, rows, acc):
    c = lax.axis_index("c")
    s = lax.axis_index("s")
    w = c * _NS + s
    _zero_rows(rows)
    _zero_acc_slice(rows, acc, s)
    plsc.subcore_barrier()
    @pl.loop(0, _CH)
    def _(i):
        for j in range(_D // 16):
            rows[i, pl.ds(j * 16, 16)] = jnp.ones((16,), jnp.float32)

    pltpu.sync_copy(dst_hbm.at[w], dst_v)

    @pl.loop(0, _NCHUNK)
    def _(j):
        pltpu.sync_copy(rows, acc.at[dst_v.at[j]], add=True)

    plsc.subcore_barrier()
    pltpu.sync_copy(
        acc.at[pl.ds(s * _RPT, _RPT)], out_hbm.at[c, pl.ds(s * _RPT, _RPT)]
    )


@functools.partial(
    pl.kernel,
    mesh=_mesh,
    out_type=jax.ShapeDtypeStruct((_NC, _ACC_ROWS, _D), jnp.float32),
    scratch_types=[
        pltpu.VMEM((_NCHUNK, _CH), jnp.int32),
        pltpu.VMEM((_NCHUNK, _CH), jnp.int32),
        pltpu.VMEM((_CH, _D), jnp.float32),
        pltpu.VMEM_SHARED((_ACC_ROWS, _D), jnp.float32),
        pltpu.SemaphoreType.DMA,
    ],
)
def _seg_kernel(hp_hbm, src_hbm, dst_hbm, out_hbm, src_v, dst_v, rows,
                acc, sem):
    c = lax.axis_index("c")
    s = lax.axis_index("s")
    w = c * _NS + s
    _zero_rows(rows)
    _zero_acc_slice(rows, acc, s)
    plsc.subcore_barrier()
    pltpu.sync_copy(dst_hbm.at[w], dst_v)
    pltpu.sync_copy(src_hbm.at[w], src_v)

    @pl.loop(0, _NCHUNK)
    def _(j):
        pltpu.async_copy(hp_hbm.at[src_v.at[j]], rows, sem).wait()
        pltpu.sync_copy(rows, acc.at[dst_v.at[j]], add=True)

    plsc.subcore_barrier()
    pltpu.sync_copy(
        acc.at[pl.ds(s * _RPT, _RPT)], out_hbm.at[c, pl.ds(s * _RPT, _RPT)]
    )


_HI = jax.lax.Precision.HIGHEST


def _tc_pre(x_ref, w1_ref, deg_ref, hp_ref, dinv_ref):
    deg = deg_ref[0, : _N, 0] + deg_ref[1, : _N, 0] + 1.0
    dinv = lax.rsqrt(deg)
    h = jnp.dot(x_ref[...], w1_ref[...], precision=_HI,
                preferred_element_type=jnp.float32)
    hp_ref[...] = h * dinv[:, None]
    dinv_ref[...] = dinv.reshape(1, _N)


def _tc_mid(acc_ref, hp_ref, dinv_ref, b1_ref, g_ref, be_ref, w2_ref, hp2_ref):
    dv = dinv_ref[0, :]
    h = (acc_ref[0, : _N] + acc_ref[1, : _N] + hp_ref[...]) * dv[:, None] + b1_ref[...]
    mean = jnp.mean(h, axis=0)
    xm = h - mean
    var = jnp.mean(xm * xm, axis=0)
    h = g_ref[...] * xm / jnp.sqrt(var + 1e-5) + be_ref[...]
    h = jnp.maximum(h, 0.0)
    h2 = jnp.dot(h, w2_ref[...], precision=_HI,
                 preferred_element_type=jnp.float32)
    hp2_ref[...] = h2 * dv[:, None]


def _tc_post(acc_ref, hp2_ref, dinv_ref, b2_ref, out_ref):
    dv = dinv_ref[0, :]
    out_ref[...] = (acc_ref[0, : _N] + acc_ref[1, : _N] + hp2_ref[...]) * dv[:, None] + b2_ref[...]


def kernel(x, edge_index, W1, b1, gamma, beta, W2, b2):
    src = edge_index[0].reshape(_NW, _EPT)
    dst = edge_index[1].reshape(_NW, _EPT)
    srcp = jnp.pad(src, ((0, 0), (0, _EPAD - _EPT))).reshape(_NW, _NCHUNK, _CH)
    dstp = jnp.pad(dst, ((0, 0), (0, _EPAD - _EPT)), constant_values=_N).reshape(
        _NW, _NCHUNK, _CH
    )

    degp = _deg_kernel(dstp)

    hp1, dinv = pl.pallas_call(
        _tc_pre,
        out_shape=[
            jax.ShapeDtypeStruct((_N, _D), jnp.float32),
            jax.ShapeDtypeStruct((1, _N), jnp.float32),
        ],
    )(x, W1, degp)

    acc1 = _seg_kernel(hp1, srcp, dstp)

    hp2 = pl.pallas_call(
        _tc_mid,
        out_shape=jax.ShapeDtypeStruct((_N, _D), jnp.float32),
    )(acc1, hp1, dinv, b1, gamma, beta, W2)

    acc2 = _seg_kernel(hp2, srcp, dstp)

    out = pl.pallas_call(
        _tc_post,
        out_shape=jax.ShapeDtypeStruct((_N, _D), jnp.float32),
    )(acc2, hp2, dinv, b2)

    return out


# speedup vs baseline: 1.0022x; 1.0022x over previous
import functools
import jax
import jax.numpy as jnp
from jax import lax
from jax.experimental import pallas as pl
from jax.experimental.pallas import tpu as pltpu
from jax.experimental.pallas import tpu_sc as plsc

_N = 10000
_E = 320000
_D = 128
_NC = 2
_NS = 16
_NW = _NC * _NS
_EPT = _E // _NW
_CH = 128
_NCHUNK = 80
_EPAD = _NCHUNK * _CH
_ACC_ROWS = 10240
_ZPT = _ACC_ROWS // _NS
_RPT = _ZPT

_mesh = plsc.VectorSubcoreMesh(core_axis_name="c", subcore_axis_name="s")


def _zero_rows(rows):
    @pl.loop(0, _CH)
    def _(i):
        for j in range(_D // 16):
            rows[i, pl.ds(j * 16, 16)] = jnp.zeros((16,), jnp.float32)


def _zero_acc_slice(rows, acc, s):
    base = s * _ZPT
    nfull = _ZPT // _CH
    for k in range(nfull):
        pltpu.sync_copy(rows, acc.at[pl.ds(base + k * _CH, _CH)])
    rem = _ZPT - nfull * _CH
    if rem:
        pltpu.sync_copy(rows.at[pl.ds(0, rem)], acc.at[pl.ds(base + nfull * _CH, rem)])


@functools.partial(
    pl.kernel,
    mesh=_mesh,
    out_type=jax.ShapeDtypeStruct((_NC, _ACC_ROWS, _D), jnp.float32),
    scratch_types=[
        pltpu.VMEM((_NCHUNK, _CH), jnp.int32),
        pltpu.VMEM((_CH, _D), jnp.float32),
        pltpu.VMEM_SHARED((_ACC_ROWS, _D), jnp.float32),
    ],
)
def _deg_kernel(dst_hbm, out_hbm, dst_v, rows, acc):
    c = lax.axis_index("c")
    s = lax.axis_index("s")
    w = c * _NS + s
    _zero_rows(rows)
    _zero_acc_slice(rows, acc, s)
    plsc.subcore_barrier()
    @pl.loop(0, _CH)
    def _(i):
        for j in range(_D // 16):
            rows[i, pl.ds(j * 16, 16)] = jnp.ones((16,), jnp.float32)

    pltpu.sync_copy(dst_hbm.at[w], dst_v)

    @pl.loop(0, _NCHUNK)
    def _(j):
        pltpu.sync_copy(rows, acc.at[dst_v.at[j]], add=True)

    plsc.subcore_barrier()
    pltpu.sync_copy(
        acc.at[pl.ds(s * _RPT, _RPT)], out_hbm.at[c, pl.ds(s * _RPT, _RPT)]
    )


@functools.partial(
    pl.kernel,
    mesh=_mesh,
    out_type=jax.ShapeDtypeStruct((_NC, _ACC_ROWS, _D), jnp.float32),
    scratch_types=[
        pltpu.VMEM((_NCHUNK, _CH), jnp.int32),
        pltpu.VMEM((_NCHUNK, _CH), jnp.int32),
        pltpu.VMEM((_CH, _D), jnp.float32),
        pltpu.VMEM_SHARED((_ACC_ROWS, _D), jnp.float32),
        pltpu.SemaphoreType.DMA,
    ],
)
def _seg_kernel(hp_hbm, src_hbm, dst_hbm, out_hbm, src_v, dst_v, rows,
                acc, sem):
    c = lax.axis_index("c")
    s = lax.axis_index("s")
    w = c * _NS + s
    _zero_rows(rows)
    _zero_acc_slice(rows, acc, s)
    plsc.subcore_barrier()
    pltpu.sync_copy(dst_hbm.at[w], dst_v)
    pltpu.sync_copy(src_hbm.at[w], src_v)

    @pl.loop(0, _NCHUNK)
    def _(j):
        pltpu.async_copy(hp_hbm.at[src_v.at[j]], rows, sem).wait()
        pltpu.sync_copy(rows, acc.at[dst_v.at[j]], add=True)

    plsc.subcore_barrier()
    pltpu.sync_copy(
        acc.at[pl.ds(s * _RPT, _RPT)], out_hbm.at[c, pl.ds(s * _RPT, _RPT)]
    )


_HI = jax.lax.Precision.HIGHEST


def _tc_pre(x_ref, w1_ref, deg_ref, hp_ref, dinv_ref):
    deg = deg_ref[0, : _N, 0] + deg_ref[1, : _N, 0] + 1.0
    dinv = lax.rsqrt(deg)
    h = jnp.dot(x_ref[...], w1_ref[...], precision=_HI,
                preferred_element_type=jnp.float32)
    hp_ref[...] = h * dinv[:, None]
    dinv_ref[...] = dinv.reshape(1, _N)


def _tc_mid(acc_ref, hp_ref, dinv_ref, b1_ref, g_ref, be_ref, w2_ref, hp2_ref):
    dv = dinv_ref[0, :]
    h = (acc_ref[0, : _N] + acc_ref[1, : _N] + hp_ref[...]) * dv[:, None] + b1_ref[...]
    mean = jnp.mean(h, axis=0)
    xm = h - mean
    var = jnp.mean(xm * xm, axis=0)
    h = g_ref[...] * xm / jnp.sqrt(var + 1e-5) + be_ref[...]
    h = jnp.maximum(h, 0.0)
    h2 = jnp.dot(h, w2_ref[...], precision=_HI,
                 preferred_element_type=jnp.float32)
    hp2_ref[...] = h2 * dv[:, None]


def _tc_post(acc_ref, hp2_ref, dinv_ref, b2_ref, out_ref):
    dv = dinv_ref[0, :]
    out_ref[...] = (acc_ref[0, : _N] + acc_ref[1, : _N] + hp2_ref[...]) * dv[:, None] + b2_ref[...]


def kernel(x, edge_index, W1, b1, gamma, beta, W2, b2):
    src = edge_index[0].reshape(_NW, _EPT)
    dst = edge_index[1].reshape(_NW, _EPT)
    srcp = jnp.pad(src, ((0, 0), (0, _EPAD - _EPT))).reshape(_NW, _NCHUNK, _CH)
    pad_n = _EPAD - _EPT
    trash = _N + (jnp.arange(pad_n, dtype=jnp.int32) % (_ACC_ROWS - _N))
    dstp = jnp.concatenate(
        [dst, jnp.broadcast_to(trash, (_NW, pad_n))], axis=1
    ).reshape(_NW, _NCHUNK, _CH)

    degp = _deg_kernel(dstp)

    hp1, dinv = pl.pallas_call(
        _tc_pre,
        out_shape=[
            jax.ShapeDtypeStruct((_N, _D), jnp.float32),
            jax.ShapeDtypeStruct((1, _N), jnp.float32),
        ],
    )(x, W1, degp)

    acc1 = _seg_kernel(hp1, srcp, dstp)

    hp2 = pl.pallas_call(
        _tc_mid,
        out_shape=jax.ShapeDtypeStruct((_N, _D), jnp.float32),
    )(acc1, hp1, dinv, b1, gamma, beta, W2)

    acc2 = _seg_kernel(hp2, srcp, dstp)

    out = pl.pallas_call(
        _tc_post,
        out_shape=jax.ShapeDtypeStruct((_N, _D), jnp.float32),
    )(acc2, hp2, dinv, b2)

    return out


# speedup vs baseline: 1.4257x; 1.4225x over previous
import functools
import jax
import jax.numpy as jnp
from jax import lax
from jax.experimental import pallas as pl
from jax.experimental.pallas import tpu as pltpu
from jax.experimental.pallas import tpu_sc as plsc

_N = 10000
_E = 320000
_D = 128
_NC = 2
_NS = 16
_NW = _NC * _NS
_EPT = _E // _NW
_CH = 128
_NCHUNK = 79
_EPAD = _NCHUNK * _CH
_ACC_ROWS = 10240
_ZPT = _ACC_ROWS // _NS
_RPT = _ZPT

_mesh = plsc.VectorSubcoreMesh(core_axis_name="c", subcore_axis_name="s")


def _zero_rows(rows):
    @pl.loop(0, _CH)
    def _(i):
        for j in range(_D // 16):
            rows[i, pl.ds(j * 16, 16)] = jnp.zeros((16,), jnp.float32)


def _zero_acc_slice(rows, acc, s):
    base = s * _ZPT
    nfull = _ZPT // _CH
    for k in range(nfull):
        pltpu.sync_copy(rows, acc.at[pl.ds(base + k * _CH, _CH)])
    rem = _ZPT - nfull * _CH
    if rem:
        pltpu.sync_copy(rows.at[pl.ds(0, rem)], acc.at[pl.ds(base + nfull * _CH, rem)])


@functools.partial(
    pl.kernel,
    mesh=_mesh,
    out_type=jax.ShapeDtypeStruct((_NC, _ACC_ROWS, _D), jnp.float32),
    scratch_types=[
        pltpu.VMEM((_NCHUNK, _CH), jnp.int32),
        pltpu.VMEM((_CH, _D), jnp.float32),
        pltpu.VMEM_SHARED((_ACC_ROWS, _D), jnp.float32),
    ],
)
def _deg_kernel(dst_hbm, out_hbm, dst_v, rows, acc):
    c = lax.axis_index("c")
    s = lax.axis_index("s")
    w = c * _NS + s
    _zero_rows(rows)
    _zero_acc_slice(rows, acc, s)
    plsc.subcore_barrier()
    @pl.loop(0, _CH)
    def _(i):
        for j in range(_D // 16):
            rows[i, pl.ds(j * 16, 16)] = jnp.ones((16,), jnp.float32)

    pltpu.sync_copy(dst_hbm.at[w], dst_v)

    @pl.loop(0, _NCHUNK)
    def _(j):
        pltpu.sync_copy(rows, acc.at[dst_v.at[j]], add=True)

    plsc.subcore_barrier()
    pltpu.sync_copy(
        acc.at[pl.ds(s * _RPT, _RPT)], out_hbm.at[c, pl.ds(s * _RPT, _RPT)]
    )


@functools.partial(
    pl.kernel,
    mesh=_mesh,
    out_type=jax.ShapeDtypeStruct((_NC, _ACC_ROWS, _D), jnp.float32),
    scratch_types=[
        pltpu.VMEM((_NCHUNK, _CH), jnp.int32),
        pltpu.VMEM((_NCHUNK, _CH), jnp.int32),
        pltpu.VMEM((_CH, _D), jnp.float32),
        pltpu.VMEM_SHARED((_ACC_ROWS, _D), jnp.float32),
        pltpu.SemaphoreType.DMA,
    ],
)
def _seg_kernel(hp_hbm, src_hbm, dst_hbm, out_hbm, src_v, dst_v, rows,
                acc, sem):
    c = lax.axis_index("c")
    s = lax.axis_index("s")
    w = c * _NS + s
    _zero_rows(rows)
    _zero_acc_slice(rows, acc, s)
    plsc.subcore_barrier()
    pltpu.sync_copy(dst_hbm.at[w], dst_v)
    pltpu.sync_copy(src_hbm.at[w], src_v)

    @pl.loop(0, _NCHUNK)
    def _(j):
        pltpu.async_copy(hp_hbm.at[src_v.at[j]], rows, sem).wait()
        pltpu.sync_copy(rows, acc.at[dst_v.at[j]], add=True)

    plsc.subcore_barrier()
    pltpu.sync_copy(
        acc.at[pl.ds(s * _RPT, _RPT)], out_hbm.at[c, pl.ds(s * _RPT, _RPT)]
    )


_HI = jax.lax.Precision.HIGHEST


def _tc_pre(x_ref, w1_ref, deg_ref, hp_ref, dinv_ref):
    deg = deg_ref[0, : _N, 0] + deg_ref[1, : _N, 0] + 1.0
    dinv = lax.rsqrt(deg)
    h = jnp.dot(x_ref[...], w1_ref[...], precision=_HI,
                preferred_element_type=jnp.float32)
    hp_ref[...] = h * dinv[:, None]
    dinv_ref[...] = dinv.reshape(1, _N)


def _tc_mid(acc_ref, hp_ref, dinv_ref, b1_ref, g_ref, be_ref, w2_ref, hp2_ref):
    dv = dinv_ref[0, :]
    h = (acc_ref[0, : _N] + acc_ref[1, : _N] + hp_ref[...]) * dv[:, None] + b1_ref[...]
    mean = jnp.mean(h, axis=0)
    xm = h - mean
    var = jnp.mean(xm * xm, axis=0)
    h = g_ref[...] * xm / jnp.sqrt(var + 1e-5) + be_ref[...]
    h = jnp.maximum(h, 0.0)
    h2 = jnp.dot(h, w2_ref[...], precision=_HI,
                 preferred_element_type=jnp.float32)
    hp2_ref[...] = h2 * dv[:, None]


def _tc_post(acc_ref, hp2_ref, dinv_ref, b2_ref, out_ref):
    dv = dinv_ref[0, :]
    out_ref[...] = (acc_ref[0, : _N] + acc_ref[1, : _N] + hp2_ref[...]) * dv[:, None] + b2_ref[...]


def kernel(x, edge_index, W1, b1, gamma, beta, W2, b2):
    src = edge_index[0].reshape(_NW, _EPT)
    dst = edge_index[1].reshape(_NW, _EPT)
    srcp = jnp.pad(src, ((0, 0), (0, _EPAD - _EPT))).reshape(_NW, _NCHUNK, _CH)
    pad_n = _EPAD - _EPT
    trash = _N + (jnp.arange(pad_n, dtype=jnp.int32) % (_ACC_ROWS - _N))
    dstp = jnp.concatenate(
        [dst, jnp.broadcast_to(trash, (_NW, pad_n))], axis=1
    ).reshape(_NW, _NCHUNK, _CH)

    degp = _deg_kernel(dstp)

    hp1, dinv = pl.pallas_call(
        _tc_pre,
        out_shape=[
            jax.ShapeDtypeStruct((_N, _D), jnp.float32),
            jax.ShapeDtypeStruct((1, _N), jnp.float32),
        ],
    )(x, W1, degp)

    acc1 = _seg_kernel(hp1, srcp, dstp)

    hp2 = pl.pallas_call(
        _tc_mid,
        out_shape=jax.ShapeDtypeStruct((_N, _D), jnp.float32),
    )(acc1, hp1, dinv, b1, gamma, beta, W2)

    acc2 = _seg_kernel(hp2, srcp, dstp)

    out = pl.pallas_call(
        _tc_post,
        out_shape=jax.ShapeDtypeStruct((_N, _D), jnp.float32),
    )(acc2, hp2, dinv, b2)

    return out


# speedup vs baseline: 1.4257x; 1.0000x over previous
import functools
import jax
import jax.numpy as jnp
from jax import lax
from jax.experimental import pallas as pl
from jax.experimental.pallas import tpu as pltpu
from jax.experimental.pallas import tpu_sc as plsc

_N = 10000
_E = 320000
_D = 128
_NC = 2
_NS = 16
_NW = _NC * _NS
_EPT = _E // _NW
_CH = 128
_NCHUNK = 79
_EPAD = _NCHUNK * _CH
_ACC_ROWS = 10240
_ZPT = _ACC_ROWS // _NS
_RPT = _ZPT

_mesh = plsc.VectorSubcoreMesh(core_axis_name="c", subcore_axis_name="s")


def _zero_rows(rows):
    @pl.loop(0, _CH)
    def _(i):
        for j in range(_D // 16):
            rows[i, pl.ds(j * 16, 16)] = jnp.zeros((16,), jnp.float32)


def _zero_acc_slice(rows, acc, s):
    base = s * _ZPT
    nfull = _ZPT // _CH
    for k in range(nfull):
        pltpu.sync_copy(rows, acc.at[pl.ds(base + k * _CH, _CH)])
    rem = _ZPT - nfull * _CH
    if rem:
        pltpu.sync_copy(rows.at[pl.ds(0, rem)], acc.at[pl.ds(base + nfull * _CH, rem)])


@functools.partial(
    pl.kernel,
    mesh=_mesh,
    out_type=jax.ShapeDtypeStruct((_NC, _ACC_ROWS, _D), jnp.float32),
    scratch_types=[
        pltpu.VMEM((_NCHUNK, _CH), jnp.int32),
        pltpu.VMEM((_CH, _D), jnp.float32),
        pltpu.VMEM_SHARED((_ACC_ROWS, _D), jnp.float32),
    ],
)
def _deg_kernel(dst_hbm, out_hbm, dst_v, rows, acc):
    c = lax.axis_index("c")
    s = lax.axis_index("s")
    w = c * _NS + s
    _zero_rows(rows)
    _zero_acc_slice(rows, acc, s)
    plsc.subcore_barrier()
    @pl.loop(0, _CH)
    def _(i):
        for j in range(_D // 16):
            rows[i, pl.ds(j * 16, 16)] = jnp.ones((16,), jnp.float32)

    pltpu.sync_copy(dst_hbm.at[w], dst_v)

    @pl.loop(0, _NCHUNK)
    def _(j):
        pltpu.sync_copy(rows, acc.at[dst_v.at[j]], add=True)

    plsc.subcore_barrier()
    pltpu.sync_copy(
        acc.at[pl.ds(s * _RPT, _RPT)], out_hbm.at[c, pl.ds(s * _RPT, _RPT)]
    )


@functools.partial(
    pl.kernel,
    mesh=_mesh,
    out_type=jax.ShapeDtypeStruct((_NC, _ACC_ROWS, _D), jnp.float32),
    scratch_types=[
        pltpu.VMEM((_NCHUNK, _CH), jnp.int32),
        pltpu.VMEM((_NCHUNK, _CH), jnp.int32),
        pltpu.VMEM((_CH, _D), jnp.float32),
        pltpu.VMEM_SHARED((_ACC_ROWS, _D), jnp.float32),
        pltpu.SemaphoreType.DMA,
    ],
)
def _seg_kernel(hp_hbm, src_hbm, dst_hbm, out_hbm, src_v, dst_v, rows,
                acc, sem):
    c = lax.axis_index("c")
    s = lax.axis_index("s")
    w = c * _NS + s
    _zero_rows(rows)
    _zero_acc_slice(rows, acc, s)
    plsc.subcore_barrier()
    pltpu.sync_copy(dst_hbm.at[w], dst_v)
    pltpu.sync_copy(src_hbm.at[w], src_v)

    @plsc.parallel_loop(0, _NCHUNK, unroll=2)
    def _(j):
        pltpu.async_copy(hp_hbm.at[src_v.at[j]], rows, sem).wait()
        pltpu.sync_copy(rows, acc.at[dst_v.at[j]], add=True)

    plsc.subcore_barrier()
    pltpu.sync_copy(
        acc.at[pl.ds(s * _RPT, _RPT)], out_hbm.at[c, pl.ds(s * _RPT, _RPT)]
    )


_HI = jax.lax.Precision.HIGHEST


def _tc_pre(x_ref, w1_ref, deg_ref, hp_ref, dinv_ref):
    deg = deg_ref[0, : _N, 0] + deg_ref[1, : _N, 0] + 1.0
    dinv = lax.rsqrt(deg)
    h = jnp.dot(x_ref[...], w1_ref[...], precision=_HI,
                preferred_element_type=jnp.float32)
    hp_ref[...] = h * dinv[:, None]
    dinv_ref[...] = dinv.reshape(1, _N)


def _tc_mid(acc_ref, hp_ref, dinv_ref, b1_ref, g_ref, be_ref, w2_ref, hp2_ref):
    dv = dinv_ref[0, :]
    h = (acc_ref[0, : _N] + acc_ref[1, : _N] + hp_ref[...]) * dv[:, None] + b1_ref[...]
    mean = jnp.mean(h, axis=0)
    xm = h - mean
    var = jnp.mean(xm * xm, axis=0)
    h = g_ref[...] * xm / jnp.sqrt(var + 1e-5) + be_ref[...]
    h = jnp.maximum(h, 0.0)
    h2 = jnp.dot(h, w2_ref[...], precision=_HI,
                 preferred_element_type=jnp.float32)
    hp2_ref[...] = h2 * dv[:, None]


def _tc_post(acc_ref, hp2_ref, dinv_ref, b2_ref, out_ref):
    dv = dinv_ref[0, :]
    out_ref[...] = (acc_ref[0, : _N] + acc_ref[1, : _N] + hp2_ref[...]) * dv[:, None] + b2_ref[...]


def kernel(x, edge_index, W1, b1, gamma, beta, W2, b2):
    src = edge_index[0].reshape(_NW, _EPT)
    dst = edge_index[1].reshape(_NW, _EPT)
    srcp = jnp.pad(src, ((0, 0), (0, _EPAD - _EPT))).reshape(_NW, _NCHUNK, _CH)
    pad_n = _EPAD - _EPT
    trash = _N + (jnp.arange(pad_n, dtype=jnp.int32) % (_ACC_ROWS - _N))
    dstp = jnp.concatenate(
        [dst, jnp.broadcast_to(trash, (_NW, pad_n))], axis=1
    ).reshape(_NW, _NCHUNK, _CH)

    degp = _deg_kernel(dstp)

    hp1, dinv = pl.pallas_call(
        _tc_pre,
        out_shape=[
            jax.ShapeDtypeStruct((_N, _D), jnp.float32),
            jax.ShapeDtypeStruct((1, _N), jnp.float32),
        ],
    )(x, W1, degp)

    acc1 = _seg_kernel(hp1, srcp, dstp)

    hp2 = pl.pallas_call(
        _tc_mid,
        out_shape=jax.ShapeDtypeStruct((_N, _D), jnp.float32),
    )(acc1, hp1, dinv, b1, gamma, beta, W2)

    acc2 = _seg_kernel(hp2, srcp, dstp)

    out = pl.pallas_call(
        _tc_post,
        out_shape=jax.ShapeDtypeStruct((_N, _D), jnp.float32),
    )(acc2, hp2, dinv, b2)

    return out
